# Initial kernel scaffold; baseline (speedup 1.0000x reference)
import jax
import jax.numpy as jnp
from jax.experimental import pallas as pl


def kernel(inputs, lengths, embed, W_ih, W_hh, b_ih, b_hh, W_out, b_out):
    raise NotImplementedError("write your pallas kernel here")


# speedup vs baseline: 1.9924x; 1.9924x over previous
import functools

import jax
import jax.numpy as jnp
from jax import lax
from jax.experimental import pallas as pl
from jax.experimental.pallas import tpu as pltpu
from jax.experimental.pallas import tpu_sc as plsc

IN_DIM = 1000
EMBED = 512
HID = 512
T = 200
B = 128
N = T * B
VPAD = 1008

def _table_body(emb_ref, w_ref, bias_ref, out_ref):
    out_ref[...] = (
        jnp.dot(emb_ref[...], w_ref[...], preferred_element_type=jnp.float32)
        + bias_ref[...]
    )


def _make_table(emb_pad, w_ih_t, bias2d):
    return pl.pallas_call(
        _table_body,
        out_shape=jax.ShapeDtypeStruct((VPAD, HID), jnp.float32),
    )(emb_pad, w_ih_t, bias2d)


_NC = 2
_NS = 16
_NW = _NC * _NS
_BPW = N // _NW
_CH = 80
_NCHUNK = _BPW // _CH

@functools.cache
def _build_gather():
    mesh = plsc.VectorSubcoreMesh(core_axis_name="c", subcore_axis_name="s")

    @functools.partial(
        pl.kernel,
        out_type=jax.ShapeDtypeStruct((N, HID), jnp.float32),
        mesh=mesh,
        scratch_types=[
            pltpu.VMEM((_BPW,), jnp.int32),
            pltpu.VMEM((_CH, HID), jnp.float32),
            pltpu.VMEM((_CH, HID), jnp.float32),
            pltpu.SemaphoreType.DMA,
            pltpu.SemaphoreType.DMA,
        ],
    )
    def _gather(table_hbm, idx_hbm, out_hbm, idx_v, buf0, buf1, sem0, sem1):
        wid = lax.axis_index("s") * _NC + lax.axis_index("c")
        base = wid * _BPW
        pltpu.sync_copy(idx_hbm.at[pl.ds(base, _BPW)], idx_v)
        bufs = (buf0, buf1)
        sems = (sem0, sem1)
        copies = [
            pltpu.async_copy(
                table_hbm.at[idx_v.at[pl.ds(0, _CH)]], bufs[0], sems[0]
            )
        ]
        for c in range(_NCHUNK):
            if c + 1 < _NCHUNK:
                copies.append(
                    pltpu.async_copy(
                        table_hbm.at[idx_v.at[pl.ds((c + 1) * _CH, _CH)]],
                        bufs[(c + 1) % 2],
                        sems[(c + 1) % 2],
                    )
                )
            copies[c].wait()
            pltpu.sync_copy(bufs[c % 2], out_hbm.at[pl.ds(base + c * _CH, _CH)])

    return _gather


def _rnn_body(len_ref, xp_ref, whh_ref, out_ref, h_ref):
    t = pl.program_id(0)

    @pl.when(t == 0)
    def _():
        h_ref[...] = jnp.zeros_like(h_ref)

    h = h_ref[...]
    h_new = jnp.maximum(
        xp_ref[0] + jnp.dot(h, whh_ref[...], preferred_element_type=jnp.float32),
        0.0,
    )
    mask = len_ref[...] > t
    h_ref[...] = jnp.where(mask, h_new, h)
    out_ref[0] = jnp.where(mask, h_new, 0.0)


def _run_rnn(lengths2d, xp3, w_hh_t):
    return pl.pallas_call(
        _rnn_body,
        grid=(T,),
        in_specs=[
            pl.BlockSpec((B, 1), lambda t: (0, 0)),
            pl.BlockSpec((1, B, HID), lambda t: (t, 0, 0)),
            pl.BlockSpec((HID, HID), lambda t: (0, 0)),
        ],
        out_specs=pl.BlockSpec((1, B, HID), lambda t: (t, 0, 0)),
        out_shape=jax.ShapeDtypeStruct((T, B, HID), jnp.float32),
        scratch_shapes=[pltpu.VMEM((B, HID), jnp.float32)],
    )(lengths2d, xp3, w_hh_t)


_RT = 256

def _proj_body(rec_ref, w_ref, b_ref, out_ref):
    logits = (
        jnp.dot(rec_ref[...], w_ref[...], preferred_element_type=jnp.float32)
        + b_ref[...]
    )
    m = jnp.max(logits, axis=-1, keepdims=True)
    e = jnp.exp(logits - m)
    s = jnp.sum(e, axis=-1, keepdims=True)
    out_ref[...] = logits - m - jnp.log(s)


def _run_proj(rec_flat, w_out_t, b_out2d):
    return pl.pallas_call(
        _proj_body,
        grid=(N // _RT,),
        in_specs=[
            pl.BlockSpec((_RT, HID), lambda i: (i, 0)),
            pl.BlockSpec((HID, IN_DIM), lambda i: (0, 0)),
            pl.BlockSpec((1, IN_DIM), lambda i: (0, 0)),
        ],
        out_specs=pl.BlockSpec((_RT, IN_DIM), lambda i: (i, 0)),
        out_shape=jax.ShapeDtypeStruct((N, IN_DIM), jnp.float32),
    )(rec_flat, w_out_t, b_out2d)


def kernel(inputs, lengths, embed, W_ih, W_hh, b_ih, b_hh, W_out, b_out):
    emb_pad = jnp.pad(embed, ((0, VPAD - (IN_DIM + 1)), (0, 0)))
    bias2d = (b_ih + b_hh).reshape(1, HID)
    table = _make_table(emb_pad, W_ih.T, bias2d)

    idx = inputs.reshape(N).astype(jnp.int32)
    xp = _build_gather()(table, idx)

    lengths2d = lengths.reshape(B, 1).astype(jnp.int32)
    rec = _run_rnn(lengths2d, xp.reshape(T, B, HID), W_hh.T)

    out_flat = _run_proj(rec.reshape(N, HID), W_out.T, b_out.reshape(1, IN_DIM))
    return out_flat.reshape(T, B, IN_DIM)
